# Initial kernel scaffold; baseline (speedup 1.0000x reference)
import jax
import jax.numpy as jnp
from jax.experimental import pallas as pl


def kernel(subgoals, W1, b1, W2, b2, history):
    raise NotImplementedError("write your pallas kernel here")


# speedup vs baseline: 1.1416x; 1.1416x over previous
import functools

import jax
import jax.numpy as jnp
from jax.experimental import pallas as pl
from jax.experimental.pallas import tpu as pltpu


def _stats_body(x_ref, o_ref):
    i = pl.program_id(0)
    x = x_ref[...]
    s1 = jnp.sum(x, axis=0)
    s2 = jnp.sum(x * x, axis=0)
    blk = jnp.concatenate(
        [s1[None, :], s2[None, :], jnp.zeros((6, x.shape[1]), jnp.float32)], axis=0
    )

    @pl.when(i == 0)
    def _():
        o_ref[...] = blk

    @pl.when(i != 0)
    def _():
        o_ref[...] = o_ref[...] + blk


def _main_body(x_ref, stats_ref, w1_ref, b1_ref, w2_ref, b2_ref, h_ref, o_ref,
               *, batch):
    bf = float(batch)
    eps_count = 1e-4
    tot = bf + eps_count
    s1 = stats_ref[0, :]
    s2 = stats_ref[1, :]
    batch_mean = s1 / bf
    batch_var = (s2 - s1 * batch_mean) / (bf - 1.0)
    mean = s1 / tot
    m2 = eps_count + batch_var * bf + batch_mean * batch_mean * (eps_count * bf / tot)
    var = m2 / tot
    inv = jax.lax.rsqrt(var + 1e-8)

    x = (x_ref[...] - mean[None, :]) * inv[None, :]
    h1 = jnp.dot(x, w1_ref[...], preferred_element_type=jnp.float32)
    h1 = jnp.maximum(h1 + b1_ref[0, :][None, :], 0.0)
    e = jnp.dot(h1, w2_ref[...], preferred_element_type=jnp.float32)
    e = e + b2_ref[0, :][None, :]

    hp = h_ref[...]
    hn = jnp.sum(hp * hp, axis=1)
    en = jnp.sum(e * e, axis=1, keepdims=True)
    g = jax.lax.dot_general(e, hp, (((1,), (1,)), ((), ())),
                            preferred_element_type=jnp.float32)
    d2 = (en + hn[None, :]) - 2.0 * g
    md = jnp.min(d2, axis=1, keepdims=True)
    o_ref[...] = jnp.sqrt(jnp.maximum(md, 1e-12))


def kernel(subgoals, W1, b1, W2, b2, history):
    B, D = subgoals.shape
    H = W1.shape[1]
    M = history.shape[0]

    MPAD = 512
    hp = jnp.concatenate(
        [history, jnp.full((MPAD - M, H), 1e15, dtype=history.dtype)], axis=0
    )
    b1r = b1.reshape(1, H)
    b2r = b2.reshape(1, H)

    SBLK = 2048
    stats = pl.pallas_call(
        _stats_body,
        grid=(B // SBLK,),
        in_specs=[pl.BlockSpec((SBLK, D), lambda i: (i, 0))],
        out_specs=pl.BlockSpec((8, D), lambda i: (0, 0)),
        out_shape=jax.ShapeDtypeStruct((8, D), jnp.float32),
        compiler_params=pltpu.CompilerParams(
            dimension_semantics=("arbitrary",)),
    )(subgoals)

    BLK = 512
    out = pl.pallas_call(
        functools.partial(_main_body, batch=B),
        grid=(B // BLK,),
        in_specs=[
            pl.BlockSpec((BLK, D), lambda i: (i, 0)),
            pl.BlockSpec((8, D), lambda i: (0, 0)),
            pl.BlockSpec((D, H), lambda i: (0, 0)),
            pl.BlockSpec((1, H), lambda i: (0, 0)),
            pl.BlockSpec((H, H), lambda i: (0, 0)),
            pl.BlockSpec((1, H), lambda i: (0, 0)),
            pl.BlockSpec((MPAD, H), lambda i: (0, 0)),
        ],
        out_specs=pl.BlockSpec((BLK, 1), lambda i: (i, 0)),
        out_shape=jax.ShapeDtypeStruct((B, 1), jnp.float32),
        compiler_params=pltpu.CompilerParams(
            dimension_semantics=("parallel",)),
    )(subgoals, stats, W1, b1r, W2, b2r, hp)

    return out


# speedup vs baseline: 1.1515x; 1.0087x over previous
import functools

import jax
import jax.numpy as jnp
from jax.experimental import pallas as pl
from jax.experimental.pallas import tpu as pltpu


def _stats_body(x_ref, o_ref):
    i = pl.program_id(0)
    x = x_ref[...]
    s1 = jnp.sum(x, axis=0)
    s2 = jnp.sum(x * x, axis=0)
    blk = jnp.concatenate(
        [s1[None, :], s2[None, :], jnp.zeros((6, x.shape[1]), jnp.float32)], axis=0
    )

    @pl.when(i == 0)
    def _():
        o_ref[...] = blk

    @pl.when(i != 0)
    def _():
        o_ref[...] = o_ref[...] + blk


def _main_body(x_ref, stats_ref, w1_ref, b1_ref, w2_ref, b2_ref, h_ref, o_ref,
               w1s_ref, w2s_ref, hps_ref, c1_ref, hn_ref, *, batch):
    i = pl.program_id(0)

    @pl.when(i == 0)
    def _():
        bf = float(batch)
        eps_count = 1e-4
        tot = bf + eps_count
        s1 = stats_ref[0, :]
        s2 = stats_ref[1, :]
        batch_mean = s1 / bf
        batch_var = (s2 - s1 * batch_mean) / (bf - 1.0)
        mean = s1 / tot
        m2 = (eps_count + batch_var * bf
              + batch_mean * batch_mean * (eps_count * bf / tot))
        var = m2 / tot
        inv = jax.lax.rsqrt(var + 1e-8)
        w1 = w1_ref[...]
        w1s_ref[...] = (w1 * inv[:, None]).astype(jnp.bfloat16)
        c1_ref[...] = b1_ref[...] - jnp.dot(
            (mean * inv)[None, :], w1, preferred_element_type=jnp.float32)
        w2s_ref[...] = w2_ref[...].astype(jnp.bfloat16)
        hp = h_ref[...]
        hn_ref[...] = jnp.sum(hp * hp, axis=1)[None, :]
        hps_ref[...] = (hp * -2.0).astype(jnp.bfloat16)

    x = x_ref[...].astype(jnp.bfloat16)
    h1 = jnp.dot(x, w1s_ref[...], preferred_element_type=jnp.float32)
    h1 = jnp.maximum(h1 + c1_ref[...], 0.0).astype(jnp.bfloat16)
    e = jnp.dot(h1, w2s_ref[...], preferred_element_type=jnp.float32)
    e = e + b2_ref[...]
    en = jnp.sum(e * e, axis=1, keepdims=True)
    gp = jax.lax.dot_general(e.astype(jnp.bfloat16), hps_ref[...],
                             (((1,), (1,)), ((), ())),
                             preferred_element_type=jnp.float32)
    t = gp + hn_ref[...]
    md = jnp.min(t, axis=1, keepdims=True) + en
    o_ref[...] = jnp.sqrt(jnp.maximum(md, 1e-12))


def kernel(subgoals, W1, b1, W2, b2, history):
    B, D = subgoals.shape
    H = W1.shape[1]
    M = history.shape[0]

    MPAD = 512
    hp = jnp.concatenate(
        [history, jnp.full((MPAD - M, H), 1e15, dtype=history.dtype)], axis=0
    )
    b1r = b1.reshape(1, H)
    b2r = b2.reshape(1, H)

    SBLK = 2048
    stats = pl.pallas_call(
        _stats_body,
        grid=(B // SBLK,),
        in_specs=[pl.BlockSpec((SBLK, D), lambda i: (i, 0))],
        out_specs=pl.BlockSpec((8, D), lambda i: (0, 0)),
        out_shape=jax.ShapeDtypeStruct((8, D), jnp.float32),
        compiler_params=pltpu.CompilerParams(
            dimension_semantics=("arbitrary",)),
    )(subgoals)

    BLK = 512
    out = pl.pallas_call(
        functools.partial(_main_body, batch=B),
        grid=(B // BLK,),
        in_specs=[
            pl.BlockSpec((BLK, D), lambda i: (i, 0)),
            pl.BlockSpec((8, D), lambda i: (0, 0)),
            pl.BlockSpec((D, H), lambda i: (0, 0)),
            pl.BlockSpec((1, H), lambda i: (0, 0)),
            pl.BlockSpec((H, H), lambda i: (0, 0)),
            pl.BlockSpec((1, H), lambda i: (0, 0)),
            pl.BlockSpec((MPAD, H), lambda i: (0, 0)),
        ],
        out_specs=pl.BlockSpec((BLK, 1), lambda i: (i, 0)),
        out_shape=jax.ShapeDtypeStruct((B, 1), jnp.float32),
        scratch_shapes=[
            pltpu.VMEM((D, H), jnp.bfloat16),
            pltpu.VMEM((H, H), jnp.bfloat16),
            pltpu.VMEM((MPAD, H), jnp.bfloat16),
            pltpu.VMEM((1, H), jnp.float32),
            pltpu.VMEM((1, H), jnp.float32),
        ],
        compiler_params=pltpu.CompilerParams(
            dimension_semantics=("arbitrary",)),
    )(subgoals, stats, W1, b1r, W2, b2r, hp)

    return out
